# Initial kernel scaffold; baseline (speedup 1.0000x reference)
import jax
import jax.numpy as jnp
from jax.experimental import pallas as pl


def kernel(h, edge_index, W_lin0, W_root0, b0, W_lin1, W_root1, b1, Wm1, bm1, Wm2, bm2):
    raise NotImplementedError("write your pallas kernel here")


# speedup vs baseline: 24.7486x; 24.7486x over previous
import functools

import jax
import jax.numpy as jnp
from jax import lax
from jax.experimental import pallas as pl
from jax.experimental.pallas import tpu as pltpu
from jax.experimental.pallas import tpu_sc as plsc

N = 10000
E = 320000
D = 128
BLK = 128
NC = 2
NS = 16
NW = NC * NS
NBLOCKS = E // BLK
NP = 10112
ROWS_PER_TILE = NP // NS
R = 1000


def _fill(ref, rows, value):
    val = jnp.full((16,), value, jnp.float32)

    def body(i, _):
        for j in range(D // 16):
            ref[i, pl.ds(j * 16, 16)] = val
        return 0

    lax.fori_loop(0, rows, body, 0, unroll=False)


def _make_sc_agg(C, with_deg):
    CT = C + (1 if with_deg else 0)
    mesh = plsc.VectorSubcoreMesh(core_axis_name="c", subcore_axis_name="s")

    @functools.partial(
        pl.kernel,
        mesh=mesh,
        out_type=jax.ShapeDtypeStruct((NC, CT, NP, D), jnp.float32),
        scratch_types=[
            pltpu.VMEM((2, BLK), jnp.int32),
            pltpu.VMEM((2, BLK), jnp.int32),
            pltpu.VMEM((2, BLK, D), jnp.float32),
            pltpu.VMEM((BLK, D), jnp.float32),
            pltpu.VMEM_SHARED((NP, D), jnp.float32),
            pltpu.SemaphoreType.DMA,
            pltpu.SemaphoreType.DMA,
        ],
    )
    def sc_agg(table, srci, dsti, out, sbuf, dbuf, rows, zbuf, acc, sem0, sem1):
        cid = lax.axis_index("c")
        sid = lax.axis_index("s")
        w = sid * NC + cid
        nb = 78 + jnp.where(w < NBLOCKS - 78 * NW, 1, 0)
        _fill(zbuf, BLK, 0.0)

        for chunk in range(CT):
            is_deg = with_deg and chunk == C
            zoff = 0
            while zoff < ROWS_PER_TILE:
                zn = min(BLK, ROWS_PER_TILE - zoff)
                pltpu.sync_copy(
                    zbuf.at[pl.ds(0, zn)],
                    acc.at[pl.ds(sid * ROWS_PER_TILE + zoff, zn)],
                )
                zoff += zn
            if is_deg:
                _fill(rows.at[0], BLK, 1.0)
            plsc.subcore_barrier()

            if is_deg:
                def body(k, _):
                    base = (w + k * NW) * BLK
                    pltpu.sync_copy(dsti.at[pl.ds(base, BLK)], dbuf.at[0])
                    pltpu.sync_copy(rows.at[0], acc.at[dbuf.at[0]], add=True)
                    return 0
            else:
                tbl = table.at[chunk]

                def body(k, _):
                    base = (w + k * NW) * BLK
                    pltpu.sync_copy(srci.at[pl.ds(base, BLK)], sbuf.at[0])
                    pltpu.sync_copy(dsti.at[pl.ds(base, BLK)], dbuf.at[0])
                    pltpu.async_copy(tbl.at[sbuf.at[0]], rows.at[0], sem0).wait()
                    pltpu.sync_copy(rows.at[0], acc.at[dbuf.at[0]], add=True)
                    return 0

            lax.fori_loop(0, nb, body, 0, unroll=False)
            plsc.subcore_barrier()

            pltpu.sync_copy(
                acc.at[pl.ds(sid * ROWS_PER_TILE, ROWS_PER_TILE)],
                out.at[cid, chunk, pl.ds(sid * ROWS_PER_TILE, ROWS_PER_TILE)],
            )

    return sc_agg


def _dg(x, w):
    return lax.dot_general(x, w, (((1,), (1,)), ((), ())),
                           preferred_element_type=jnp.float32)


def _dinv_from(g_ref):
    deg = g_ref[0, 0, :, 0:1] + g_ref[1, 0, :, 0:1]
    return jnp.where(deg > 0, 1.0 / deg, 0.0)


def _tc1_body(h_ref, s_ref, g_ref, wl_ref, wr_ref, b_ref, out_ref):
    s0 = s_ref[0, 0] + s_ref[1, 0]
    agg = s0 * _dinv_from(g_ref)
    y = _dg(agg, wl_ref[...]) + _dg(h_ref[0], wr_ref[...]) + b_ref[...]
    y = jnp.maximum(y, 0.0)
    out_ref[0, 0] = y[:, :D]
    out_ref[0, 1] = y[:, D:]


def _tc2_body(x1_ref, s_ref, g_ref, wl_ref, wr_ref, b_ref,
              wm1_ref, bm1_ref, wm2_ref, bm2_ref, out_ref):
    dinv = _dinv_from(g_ref)
    s1 = jnp.concatenate(
        [s_ref[0, 0] + s_ref[1, 0], s_ref[0, 1] + s_ref[1, 1]], axis=1)
    x1 = jnp.concatenate([x1_ref[0, 0], x1_ref[0, 1]], axis=1)
    x2 = _dg(s1 * dinv, wl_ref[...]) + _dg(x1, wr_ref[...]) + b_ref[...]
    x2 = jnp.maximum(x2, 0.0)
    x3 = jnp.maximum(_dg(x2, wm1_ref[...]) + bm1_ref[...], 0.0)
    out_ref[0] = _dg(x3, wm2_ref[...]) + bm2_ref[...]


def kernel(h, edge_index, W_lin0, W_root0, b0, W_lin1, W_root1, b1,
           Wm1, bm1, Wm2, bm2):
    B = h.shape[0]
    HOR = Wm2.shape[0]
    src = edge_index[0]
    dst = edge_index[1]

    s0p = _make_sc_agg(B, with_deg=True)(h, src, dst)

    grid = (B, N // R)
    full = lambda s: pl.BlockSpec(s, lambda b, i: (0,) * len(s))
    deg_spec = pl.BlockSpec((NC, 1, R, D), lambda b, i: (0, B, i, 0))

    x1 = pl.pallas_call(
        _tc1_body,
        grid=grid,
        in_specs=[
            pl.BlockSpec((1, R, D), lambda b, i: (b, i, 0)),
            pl.BlockSpec((NC, 1, R, D), lambda b, i: (0, b, i, 0)),
            deg_spec,
            full(W_lin0.shape),
            full(W_root0.shape),
            full((1, 2 * D)),
        ],
        out_specs=pl.BlockSpec((1, 2, R, D), lambda b, i: (b, 0, i, 0)),
        out_shape=jax.ShapeDtypeStruct((B, 2, N, D), jnp.float32),
    )(h, s0p, s0p, W_lin0, W_root0, b0.reshape(1, -1))

    s1p = _make_sc_agg(2 * B, with_deg=False)(
        x1.reshape(2 * B, N, D), src, dst)

    y = pl.pallas_call(
        _tc2_body,
        grid=grid,
        in_specs=[
            pl.BlockSpec((1, 2, R, D), lambda b, i: (b, 0, i, 0)),
            pl.BlockSpec((NC, 2, R, D), lambda b, i: (0, b, i, 0)),
            deg_spec,
            full(W_lin1.shape),
            full(W_root1.shape),
            full((1, 2 * D)),
            full(Wm1.shape),
            full((1, 2 * D)),
            full(Wm2.shape),
            full((1, HOR)),
        ],
        out_specs=pl.BlockSpec((1, R, HOR), lambda b, i: (b, i, 0)),
        out_shape=jax.ShapeDtypeStruct((B, N, HOR), jnp.float32),
    )(x1, s1p, s0p, W_lin1, W_root1, b1.reshape(1, -1),
      Wm1, bm1.reshape(1, -1), Wm2, bm2.reshape(1, -1))

    return jnp.transpose(y.reshape(B, N, HOR, 1), (0, 2, 1, 3))
